# baseline (speedup 1.0000x reference)
import jax
import jax.numpy as jnp
from jax import lax
from jax.experimental import pallas as pl
from jax.experimental.pallas import tpu as pltpu
from jax.experimental.pallas import tpu_sc as plsc

DIM = 32
NBUF = 4
NUM_CORES = 2
NUM_SUBCORES = 16
NUM_WORKERS = NUM_CORES * NUM_SUBCORES


def _gather_body(table_hbm, idx_hbm, out_hbm, idx_v, rows_v, sem_g, sem_s):
    wid = lax.axis_index("s") * NUM_CORES + lax.axis_index("c")
    per_b = idx_v.shape[0]
    hist = idx_v.shape[1]
    base = wid * per_b
    pltpu.sync_copy(idx_hbm.at[pl.ds(base, per_b)], idx_v)

    def fire_gather(i):
        buf = lax.rem(i, NBUF)
        pltpu.async_copy(
            table_hbm.at[idx_v.at[i]],
            rows_v.at[buf],
            sem_g,
        )

    def wait_gather(i):
        buf = lax.rem(i, NBUF)
        pltpu.make_async_copy(
            table_hbm.at[idx_v.at[i]],
            rows_v.at[buf],
            sem_g,
        ).wait()

    def fire_store(i):
        buf = lax.rem(i, NBUF)
        pltpu.async_copy(rows_v.at[buf], out_hbm.at[base + i], sem_s)

    def wait_store(i):
        buf = lax.rem(i, NBUF)
        pltpu.make_async_copy(
            rows_v.at[buf], out_hbm.at[base + i], sem_s
        ).wait()

    for i in range(NBUF - 1):
        fire_gather(jnp.int32(i))

    def body(i, carry):
        wait_gather(i)
        fire_store(i)

        @pl.when(i >= 1)
        def _():
            wait_store(i - 1)

        @pl.when(i + (NBUF - 1) < per_b)
        def _():
            fire_gather(i + (NBUF - 1))

        return carry

    lax.fori_loop(0, per_b, body, 0)
    wait_store(jnp.int32(per_b - 1))


def kernel(entities, table):
    b, h = entities.shape
    idx = entities.astype(jnp.int32)
    per_b = b // NUM_WORKERS

    eye = jnp.eye(DIM, dtype=jnp.float32)
    table_lin = jax.lax.dot_general(
        table,
        eye,
        (((1,), (0,)), ((), ())),
        precision=jax.lax.Precision.HIGHEST,
    )

    mesh = plsc.VectorSubcoreMesh(core_axis_name="c", subcore_axis_name="s")
    out = pl.kernel(
        _gather_body,
        out_type=jax.ShapeDtypeStruct((b, h, DIM), jnp.float32),
        mesh=mesh,
        scratch_types=[
            pltpu.VMEM((per_b, h), jnp.int32),
            pltpu.VMEM((NBUF, h, DIM), jnp.float32),
            pltpu.SemaphoreType.DMA,
            pltpu.SemaphoreType.DMA,
        ],
        compiler_params=pltpu.CompilerParams(use_tc_tiling_on_sc=False),
    )(table_lin, idx)
    return jax.lax.dot_general(
        out,
        eye,
        (((2,), (0,)), ((), ())),
        precision=jax.lax.Precision.HIGHEST,
    )


# speedup vs baseline: 1.5138x; 1.5138x over previous
import jax
import jax.numpy as jnp
from jax import lax
from jax.experimental import pallas as pl
from jax.experimental.pallas import tpu as pltpu
from jax.experimental.pallas import tpu_sc as plsc

DIM = 32
NBUF = 4
ROWS_PER_BUF = 4
NUM_CORES = 2
NUM_SUBCORES = 16
NUM_WORKERS = NUM_CORES * NUM_SUBCORES


def _gather_body(table_hbm, idx_hbm, out_hbm, idx_v, rows_v, sem_g, sem_s):
    wid = lax.axis_index("s") * NUM_CORES + lax.axis_index("c")
    per_b = idx_v.shape[0]
    n = per_b // ROWS_PER_BUF
    base = wid * per_b
    pltpu.sync_copy(idx_hbm.at[pl.ds(base, per_b)], idx_v)

    def fire_gathers(i):
        buf = lax.rem(i, NBUF)
        for r in range(ROWS_PER_BUF):
            pltpu.async_copy(
                table_hbm.at[idx_v.at[i * ROWS_PER_BUF + r]],
                rows_v.at[buf, r],
                sem_g,
            )

    def wait_gathers(i):
        buf = lax.rem(i, NBUF)
        for r in range(ROWS_PER_BUF):
            pltpu.make_async_copy(
                table_hbm.at[idx_v.at[i * ROWS_PER_BUF + r]],
                rows_v.at[buf, r],
                sem_g,
            ).wait()

    def fire_store(i):
        buf = lax.rem(i, NBUF)
        pltpu.async_copy(
            rows_v.at[buf],
            out_hbm.at[pl.ds(base + i * ROWS_PER_BUF, ROWS_PER_BUF)],
            sem_s,
        )

    def wait_store(i):
        buf = lax.rem(i, NBUF)
        pltpu.make_async_copy(
            rows_v.at[buf],
            out_hbm.at[pl.ds(base + i * ROWS_PER_BUF, ROWS_PER_BUF)],
            sem_s,
        ).wait()

    for i in range(NBUF - 1):
        fire_gathers(jnp.int32(i))

    def body(i, carry):
        wait_gathers(i)
        fire_store(i)

        @pl.when(i >= 1)
        def _():
            wait_store(i - 1)

        @pl.when(i + (NBUF - 1) < n)
        def _():
            fire_gathers(i + (NBUF - 1))

        return carry

    lax.fori_loop(0, n, body, 0)
    wait_store(jnp.int32(n - 1))


def kernel(entities, table):
    b, h = entities.shape
    idx = entities.astype(jnp.int32)
    per_b = b // NUM_WORKERS

    mesh = plsc.VectorSubcoreMesh(core_axis_name="c", subcore_axis_name="s")
    out = pl.kernel(
        _gather_body,
        out_type=jax.ShapeDtypeStruct((b, h, DIM), jnp.float32),
        mesh=mesh,
        scratch_types=[
            pltpu.VMEM((per_b, h), jnp.int32),
            pltpu.VMEM((NBUF, ROWS_PER_BUF, h, DIM), jnp.float32),
            pltpu.SemaphoreType.DMA,
            pltpu.SemaphoreType.DMA,
        ],
        compiler_params=pltpu.CompilerParams(use_tc_tiling_on_sc=False),
    )(table, idx)
    return out


# speedup vs baseline: 1.5153x; 1.0010x over previous
import jax
import jax.numpy as jnp
from jax import lax
from jax.experimental import pallas as pl
from jax.experimental.pallas import tpu as pltpu
from jax.experimental.pallas import tpu_sc as plsc

DIM = 32
NBUF = 6
ROWS_PER_BUF = 8
NUM_CORES = 2
NUM_SUBCORES = 16
NUM_WORKERS = NUM_CORES * NUM_SUBCORES


def _gather_body(table_hbm, idx_hbm, out_hbm, idx_v, rows_v, sem_g, sem_s):
    wid = lax.axis_index("s") * NUM_CORES + lax.axis_index("c")
    per_b = idx_v.shape[0]
    n = per_b // ROWS_PER_BUF
    base = wid * per_b
    pltpu.sync_copy(idx_hbm.at[pl.ds(base, per_b)], idx_v)

    def fire_gathers(i):
        buf = lax.rem(i, NBUF)
        for r in range(ROWS_PER_BUF):
            pltpu.async_copy(
                table_hbm.at[idx_v.at[i * ROWS_PER_BUF + r]],
                rows_v.at[buf, r],
                sem_g,
            )

    def wait_gathers(i):
        buf = lax.rem(i, NBUF)
        for r in range(ROWS_PER_BUF):
            pltpu.make_async_copy(
                table_hbm.at[idx_v.at[i * ROWS_PER_BUF + r]],
                rows_v.at[buf, r],
                sem_g,
            ).wait()

    def fire_store(i):
        buf = lax.rem(i, NBUF)
        pltpu.async_copy(
            rows_v.at[buf],
            out_hbm.at[pl.ds(base + i * ROWS_PER_BUF, ROWS_PER_BUF)],
            sem_s,
        )

    def wait_store(i):
        buf = lax.rem(i, NBUF)
        pltpu.make_async_copy(
            rows_v.at[buf],
            out_hbm.at[pl.ds(base + i * ROWS_PER_BUF, ROWS_PER_BUF)],
            sem_s,
        ).wait()

    for i in range(NBUF - 1):
        fire_gathers(jnp.int32(i))

    def body(i, carry):
        wait_gathers(i)
        fire_store(i)

        @pl.when(i >= 1)
        def _():
            wait_store(i - 1)

        @pl.when(i + (NBUF - 1) < n)
        def _():
            fire_gathers(i + (NBUF - 1))

        return carry

    lax.fori_loop(0, n, body, 0)
    wait_store(jnp.int32(n - 1))


def kernel(entities, table):
    b, h = entities.shape
    idx = entities.astype(jnp.int32)
    per_b = b // NUM_WORKERS

    mesh = plsc.VectorSubcoreMesh(core_axis_name="c", subcore_axis_name="s")
    out = pl.kernel(
        _gather_body,
        out_type=jax.ShapeDtypeStruct((b, h, DIM), jnp.float32),
        mesh=mesh,
        scratch_types=[
            pltpu.VMEM((per_b, h), jnp.int32),
            pltpu.VMEM((NBUF, ROWS_PER_BUF, h, DIM), jnp.float32),
            pltpu.SemaphoreType.DMA,
            pltpu.SemaphoreType.DMA,
        ],
        compiler_params=pltpu.CompilerParams(use_tc_tiling_on_sc=False),
    )(table, idx)
    return out
